# baseline (speedup 1.0000x reference)
import functools

import jax
import jax.numpy as jnp
from jax import lax
from jax.experimental import pallas as pl
from jax.experimental.pallas import tpu as pltpu
from jax.experimental.pallas import tpu_sc as plsc

_NC = 2
_NS = 16
_NW = _NC * _NS


def _make_sc_kernel(n, V, D, SYN_V):
    rpw = n // _NW
    chunks = rpw // 8
    zero_iters = (rpw * SYN_V) // 128
    mesh = plsc.VectorSubcoreMesh(core_axis_name="c", subcore_axis_name="s")

    @functools.partial(
        pl.kernel,
        mesh=mesh,
        out_type=[
            jax.ShapeDtypeStruct((n, D), jnp.float32),
            jax.ShapeDtypeStruct((n, D), jnp.float32),
            jax.ShapeDtypeStruct((n * SYN_V,), jnp.float32),
        ],
        scratch_types=[
            pltpu.VMEM((rpw,), jnp.int32),
            pltpu.VMEM((rpw, D), jnp.float32),
            pltpu.VMEM((rpw, D), jnp.float32),
            pltpu.VMEM((2 * rpw,), jnp.int32),
            pltpu.VMEM((2 * rpw,), jnp.int32),
            pltpu.VMEM((rpw * SYN_V,), jnp.float32),
            pltpu.SemaphoreType.DMA,
            pltpu.SemaphoreType.DMA,
            pltpu.SemaphoreType.DMA,
        ],
        compiler_params=pltpu.CompilerParams(needs_layout_passes=False),
    )
    def sc_kernel(ids_hbm, syn_flat_hbm, t1_hbm, t2_hbm,
                  g1_hbm, g2_hbm, sis_hbm,
                  idx_v, g1_v, g2_v, idx2_v, syn_v, sis_v, sem1, sem2, sem3):
        wid = lax.axis_index("s") * _NC + lax.axis_index("c")
        base = wid * rpw
        lane = lax.broadcasted_iota(jnp.int32, (16,), 0)
        pltpu.sync_copy(ids_hbm.at[pl.ds(base, rpw)], idx_v)
        cp1 = pltpu.async_copy(t1_hbm.at[idx_v], g1_v, sem1)
        cp2 = pltpu.async_copy(t2_hbm.at[idx_v], g2_v, sem2)
        for c in range(rpw // 16):
            v2 = idx_v[pl.ds(c * 16, 16)] * 2
            pos = lane * 2 + (c * 32)
            plsc.store_scatter(idx2_v, [pos], v2)
            plsc.store_scatter(idx2_v, [pos + 1], v2 + 1)
        cp3 = pltpu.async_copy(syn_flat_hbm.at[idx2_v], syn_v, sem3)

        zeros16 = jnp.zeros((16,), jnp.float32)

        def zero_body(i, carry):
            off = pl.multiple_of(i * 128, 128)
            for u in range(8):
                sis_v[pl.ds(off + u * 16, 16)] = zeros16
            return carry

        lax.fori_loop(0, zero_iters, zero_body, 0)

        cp3.wait()
        rows0 = lax.shift_right_logical(lane, 1)
        cols = lax.bitwise_and(lane, 1)
        even = cols == 0
        odd = cols == 1
        ones16 = jnp.ones((16,), jnp.float32)
        for c in range(chunks):
            vals = syn_v[pl.ds(c * 16, 16)]
            flat = (rows0 + 8 * c) * SYN_V + vals
            plsc.addupdate_scatter(sis_v, [flat], ones16, mask=even)
            plsc.addupdate_scatter(sis_v, [flat], ones16, mask=odd)

        cp1.wait()
        cp2.wait()
        pltpu.sync_copy(g1_v, g1_hbm.at[pl.ds(base, rpw)])
        pltpu.sync_copy(g2_v, g2_hbm.at[pl.ds(base, rpw)])
        pltpu.sync_copy(sis_v, sis_hbm.at[pl.ds(base * SYN_V, rpw * SYN_V)])

    return sc_kernel


def _tc_body(g1_ref, g2_ref, wemb_ref, bemb_ref, wsum_ref, bsum_ref, pad_ref,
             wrev_ref, emb_ref, sie_ref, ess_ref, x_ref):
    @pl.when(pl.program_id(0) == 0)
    def _():
        g2 = g2_ref[...]
        sie = jax.nn.sigmoid(g2 - 4.0) - jax.nn.sigmoid(-g2 - 4.0)
        sie_ref[...] = sie
        ess_ref[...] = (
            jnp.dot(sie, wsum_ref[...], preferred_element_type=jnp.float32)
            + bsum_ref[...]
        )
        x = (
            g1_ref[...]
            + jnp.dot(sie, wemb_ref[...], preferred_element_type=jnp.float32)
            + bemb_ref[...]
        )
        x_ref[...] = jnp.concatenate([x, pad_ref[...]], axis=1)

    emb_ref[...] = lax.dot_general(
        x_ref[...], wrev_ref[...],
        (((1,), (1,)), ((), ())),
        preferred_element_type=jnp.float32,
    )


def _make_tc_kernel(n, V, D, ACD, SYN_V, BN):
    nblk = pl.cdiv(V, BN)
    emb_d = D + ACD
    return pl.pallas_call(
        _tc_body,
        grid=(nblk,),
        in_specs=[
            pl.BlockSpec((n, D), lambda j: (0, 0)),
            pl.BlockSpec((n, D), lambda j: (0, 0)),
            pl.BlockSpec((D, D), lambda j: (0, 0)),
            pl.BlockSpec((1, D), lambda j: (0, 0)),
            pl.BlockSpec((D, SYN_V), lambda j: (0, 0)),
            pl.BlockSpec((1, SYN_V), lambda j: (0, 0)),
            pl.BlockSpec((n, ACD), lambda j: (0, 0)),
            pl.BlockSpec((BN, emb_d), lambda j: (j, 0)),
        ],
        out_specs=[
            pl.BlockSpec((n, BN), lambda j: (0, j)),
            pl.BlockSpec((n, D), lambda j: (0, 0)),
            pl.BlockSpec((n, SYN_V), lambda j: (0, 0)),
        ],
        out_shape=[
            jax.ShapeDtypeStruct((n, V), jnp.float32),
            jax.ShapeDtypeStruct((n, D), jnp.float32),
            jax.ShapeDtypeStruct((n, SYN_V), jnp.float32),
        ],
        scratch_shapes=[pltpu.VMEM((n, emb_d), jnp.float32)],
        compiler_params=pltpu.CompilerParams(
            vmem_limit_bytes=100 * 1024 * 1024,
        ),
    )


def kernel(ids, syn_map, table1, table2, W_emb_out, b_emb_out,
           W_sum_out, b_sum_out, W_rev, padding):
    n = ids.shape[0]
    V, D = table1.shape
    SYN_V = W_sum_out.shape[1]
    ACD = padding.shape[1]

    ids = ids.astype(jnp.int32)
    syn_map = syn_map.astype(jnp.int32)

    sc = _make_sc_kernel(n, V, D, SYN_V)
    g1, g2, sis_flat = sc(ids, syn_map.reshape(-1), table1, table2)

    tc = _make_tc_kernel(n, V, D, ACD, SYN_V, 2048)
    emb, sie, ess = tc(
        g1, g2, W_emb_out, b_emb_out.reshape(1, D),
        W_sum_out, b_sum_out.reshape(1, SYN_V), padding, W_rev,
    )
    return emb, sie, sis_flat.reshape(n, SYN_V), ess


# speedup vs baseline: 3.5111x; 3.5111x over previous
import functools

import jax
import jax.numpy as jnp
from jax import lax
from jax.experimental import pallas as pl
from jax.experimental.pallas import tpu as pltpu
from jax.experimental.pallas import tpu_sc as plsc

_NC = 2
_NS = 16
_NW = _NC * _NS


def _make_sc_kernel(n, V, D, SYN_V):
    rpw = n // _NW
    zero_iters = (rpw * SYN_V) // 128
    mesh = plsc.VectorSubcoreMesh(core_axis_name="c", subcore_axis_name="s")

    @functools.partial(
        pl.kernel,
        mesh=mesh,
        out_type=[
            jax.ShapeDtypeStruct((n, D), jnp.float32),
            jax.ShapeDtypeStruct((n, D), jnp.float32),
            jax.ShapeDtypeStruct((n * SYN_V,), jnp.float32),
        ],
        scratch_types=[
            pltpu.VMEM((rpw,), jnp.int32),
            pltpu.VMEM((rpw, D), jnp.float32),
            pltpu.VMEM((rpw, D), jnp.float32),
            pltpu.VMEM((rpw,), jnp.int32),
            pltpu.VMEM((rpw,), jnp.int32),
            pltpu.VMEM((rpw * SYN_V,), jnp.float32),
            pltpu.SemaphoreType.DMA,
            pltpu.SemaphoreType.DMA,
            pltpu.SemaphoreType.DMA,
        ],
        compiler_params=pltpu.CompilerParams(needs_layout_passes=False),
    )
    def sc_kernel(ids_hbm, syn0_hbm, syn1_hbm, t1_hbm, t2_hbm,
                  g1_hbm, g2_hbm, sis_hbm,
                  idx_v, g1_v, g2_v, syn0_v, syn1_v, sis_v, sem1, sem2, sem3):
        wid = lax.axis_index("s") * _NC + lax.axis_index("c")
        base = wid * rpw
        lane = lax.broadcasted_iota(jnp.int32, (16,), 0)
        pltpu.sync_copy(ids_hbm.at[pl.ds(base, rpw)], idx_v)
        cp1 = pltpu.async_copy(t1_hbm.at[idx_v], g1_v, sem1)
        cp2 = pltpu.async_copy(t2_hbm.at[idx_v], g2_v, sem2)
        cp3a = pltpu.async_copy(syn0_hbm.at[idx_v], syn0_v, sem3)
        cp3b = pltpu.async_copy(syn1_hbm.at[idx_v], syn1_v, sem3)

        zeros16 = jnp.zeros((16,), jnp.float32)

        def zero_body(i, carry):
            off = pl.multiple_of(i * 128, 128)
            for u in range(8):
                sis_v[pl.ds(off + u * 16, 16)] = zeros16
            return carry

        lax.fori_loop(0, zero_iters, zero_body, 0)

        cp3a.wait()
        cp3b.wait()
        ones16 = jnp.ones((16,), jnp.float32)
        for c in range(rpw // 16):
            rows = lane + 16 * c
            flat0 = rows * SYN_V + syn0_v[pl.ds(c * 16, 16)]
            plsc.addupdate_scatter(sis_v, [flat0], ones16)
            flat1 = rows * SYN_V + syn1_v[pl.ds(c * 16, 16)]
            plsc.addupdate_scatter(sis_v, [flat1], ones16)

        cp1.wait()
        cp2.wait()
        pltpu.sync_copy(g1_v, g1_hbm.at[pl.ds(base, rpw)])
        pltpu.sync_copy(g2_v, g2_hbm.at[pl.ds(base, rpw)])
        pltpu.sync_copy(sis_v, sis_hbm.at[pl.ds(base * SYN_V, rpw * SYN_V)])

    return sc_kernel


def _tc_body(g1_ref, g2_ref, wemb_ref, bemb_ref, wsum_ref, bsum_ref, pad_ref,
             wrevt_ref, embt_ref, sie_ref, ess_ref, x_ref):
    @pl.when(pl.program_id(0) == 0)
    def _():
        g2 = g2_ref[...]
        sie = jax.nn.sigmoid(g2 - 4.0) - jax.nn.sigmoid(-g2 - 4.0)
        sie_ref[...] = sie
        ess_ref[...] = (
            jnp.dot(sie, wsum_ref[...], preferred_element_type=jnp.float32)
            + bsum_ref[...]
        )
        x = (
            g1_ref[...]
            + jnp.dot(sie, wemb_ref[...], preferred_element_type=jnp.float32)
            + bemb_ref[...]
        )
        x_ref[...] = jnp.concatenate([x, pad_ref[...]], axis=1)

    embt_ref[...] = lax.dot_general(
        wrevt_ref[...], x_ref[...],
        (((0,), (1,)), ((), ())),
        preferred_element_type=jnp.float32,
    )


def _make_tc_kernel(n, V, D, ACD, SYN_V, BN):
    nblk = pl.cdiv(V, BN)
    emb_d = D + ACD
    return pl.pallas_call(
        _tc_body,
        grid=(nblk,),
        in_specs=[
            pl.BlockSpec((n, D), lambda j: (0, 0)),
            pl.BlockSpec((n, D), lambda j: (0, 0)),
            pl.BlockSpec((D, D), lambda j: (0, 0)),
            pl.BlockSpec((1, D), lambda j: (0, 0)),
            pl.BlockSpec((D, SYN_V), lambda j: (0, 0)),
            pl.BlockSpec((1, SYN_V), lambda j: (0, 0)),
            pl.BlockSpec((n, ACD), lambda j: (0, 0)),
            pl.BlockSpec((emb_d, BN), lambda j: (0, j)),
        ],
        out_specs=[
            pl.BlockSpec((BN, n), lambda j: (j, 0)),
            pl.BlockSpec((n, D), lambda j: (0, 0)),
            pl.BlockSpec((n, SYN_V), lambda j: (0, 0)),
        ],
        out_shape=[
            jax.ShapeDtypeStruct((V, n), jnp.float32),
            jax.ShapeDtypeStruct((n, D), jnp.float32),
            jax.ShapeDtypeStruct((n, SYN_V), jnp.float32),
        ],
        scratch_shapes=[pltpu.VMEM((n, emb_d), jnp.float32)],
        compiler_params=pltpu.CompilerParams(
            vmem_limit_bytes=100 * 1024 * 1024,
            fuse_transposed_lhs_in_matmul=True,
        ),
    )


def kernel(ids, syn_map, table1, table2, W_emb_out, b_emb_out,
           W_sum_out, b_sum_out, W_rev, padding):
    n = ids.shape[0]
    V, D = table1.shape
    SYN_V = W_sum_out.shape[1]
    ACD = padding.shape[1]

    ids = ids.astype(jnp.int32)
    syn_map = syn_map.astype(jnp.int32)

    sc = _make_sc_kernel(n, V, D, SYN_V)
    g1, g2, sis_flat = sc(ids, syn_map[:, 0], syn_map[:, 1], table1, table2)

    tc = _make_tc_kernel(n, V, D, ACD, SYN_V, 2048)
    embt, sie, ess = tc(
        g1, g2, W_emb_out, b_emb_out.reshape(1, D),
        W_sum_out, b_sum_out.reshape(1, SYN_V), padding, W_rev.T,
    )
    return embt.T, sie, sis_flat.reshape(n, SYN_V), ess


# speedup vs baseline: 3.7267x; 1.0614x over previous
import functools

import jax
import jax.numpy as jnp
from jax import lax
from jax.experimental import pallas as pl
from jax.experimental.pallas import tpu as pltpu
from jax.experimental.pallas import tpu_sc as plsc

_NC = 2
_NS = 16
_NW = _NC * _NS


def _make_sc_kernel(n, V, D, SYN_V):
    rpw = n // _NW
    mesh = plsc.VectorSubcoreMesh(core_axis_name="c", subcore_axis_name="s")

    @functools.partial(
        pl.kernel,
        mesh=mesh,
        out_type=[
            jax.ShapeDtypeStruct((n, D), jnp.float32),
            jax.ShapeDtypeStruct((n, D), jnp.float32),
            jax.ShapeDtypeStruct((n, SYN_V), jnp.float32),
        ],
        scratch_types=[
            pltpu.VMEM((rpw,), jnp.int32),
            pltpu.VMEM((rpw,), jnp.int32),
            pltpu.VMEM((rpw, D), jnp.float32),
            pltpu.VMEM((rpw, D), jnp.float32),
            pltpu.VMEM((rpw,), jnp.int32),
            pltpu.VMEM((rpw,), jnp.int32),
            pltpu.VMEM((rpw, SYN_V), jnp.float32),
            pltpu.SemaphoreType.DMA,
            pltpu.SemaphoreType.DMA,
            pltpu.SemaphoreType.DMA,
        ],
        compiler_params=pltpu.CompilerParams(needs_layout_passes=False),
    )
    def sc_kernel(ids_hbm, syn_flat_hbm, t1_hbm, t2_hbm,
                  g1_hbm, g2_hbm, sist_hbm,
                  idx_v, idxo_v, g1_v, g2_v, syn0_v, syn1_v, sis_v,
                  sem1, sem2, sem3):
        wid = lax.axis_index("s") * _NC + lax.axis_index("c")
        base = wid * rpw
        lane = lax.broadcasted_iota(jnp.int32, (16,), 0)
        pltpu.sync_copy(ids_hbm.at[pl.ds(base, rpw)], idx_v)
        cp1 = pltpu.async_copy(t1_hbm.at[idx_v], g1_v, sem1)
        cp2 = pltpu.async_copy(t2_hbm.at[idx_v], g2_v, sem2)
        cp3a = pltpu.async_copy(syn_flat_hbm.at[idx_v], syn0_v, sem3)
        for c in range(rpw // 16):
            idxo_v[pl.ds(c * 16, 16)] = idx_v[pl.ds(c * 16, 16)] + V
        cp3b = pltpu.async_copy(syn_flat_hbm.at[idxo_v], syn1_v, sem3)

        zeros16 = jnp.zeros((16,), jnp.float32)
        n_chunks = SYN_V // 16
        tail = SYN_V - 16 * n_chunks
        offs = [u * 16 for u in range(n_chunks)] + ([SYN_V - 16] if tail else [])

        def zero_body(r, carry):
            for off in offs:
                sis_v[r, pl.ds(off, 16)] = zeros16
            return carry

        lax.fori_loop(0, rpw, zero_body, 0)

        cp3a.wait()
        cp3b.wait()
        ones16 = jnp.ones((16,), jnp.float32)
        for c in range(rpw // 16):
            rows = lane + 16 * c
            plsc.addupdate_scatter(sis_v, [rows, syn0_v[pl.ds(c * 16, 16)]], ones16)
            plsc.addupdate_scatter(sis_v, [rows, syn1_v[pl.ds(c * 16, 16)]], ones16)

        cp1.wait()
        cp2.wait()
        pltpu.sync_copy(g1_v, g1_hbm.at[pl.ds(base, rpw)])
        pltpu.sync_copy(g2_v, g2_hbm.at[pl.ds(base, rpw)])
        pltpu.sync_copy(sis_v, sist_hbm.at[pl.ds(base, rpw), :])

    return sc_kernel


def _tc_body(nblk, g1_ref, g2_ref, wemb_ref, bemb_ref, wsumt_ref, bsumt_ref,
             wrevt_ref, embt_ref, sie_ref, esst_ref, x_ref):
    @pl.when(pl.program_id(0) == 0)
    def _():
        g2 = g2_ref[...]
        sie = jax.nn.sigmoid(g2 - 4.0) - jax.nn.sigmoid(-g2 - 4.0)
        sie_ref[...] = sie
        x = (
            g1_ref[...]
            + jnp.dot(sie, wemb_ref[...], preferred_element_type=jnp.float32)
            + bemb_ref[...]
        )
        pad = jnp.full((x.shape[0], 64), 0.1, dtype=jnp.float32)
        x_ref[...] = jnp.concatenate([x, pad], axis=1)

    embt_ref[...] = lax.dot_general(
        wrevt_ref[...], x_ref[...],
        (((0,), (1,)), ((), ())),
        preferred_element_type=jnp.float32,
    )

    @pl.when(pl.program_id(0) == nblk - 1)
    def _():
        esst_ref[...] = (
            lax.dot_general(
                wsumt_ref[...], sie_ref[...],
                (((1,), (1,)), ((), ())),
                preferred_element_type=jnp.float32,
            )
            + bsumt_ref[...]
        )


def _make_tc_kernel(n, V, D, ACD, SYN_V, BN):
    nblk = pl.cdiv(V, BN)
    emb_d = D + ACD
    return pl.pallas_call(
        functools.partial(_tc_body, nblk),
        grid=(nblk,),
        in_specs=[
            pl.BlockSpec((n, D), lambda j: (0, 0)),
            pl.BlockSpec((n, D), lambda j: (0, 0)),
            pl.BlockSpec((D, D), lambda j: (0, 0)),
            pl.BlockSpec((1, D), lambda j: (0, 0)),
            pl.BlockSpec((SYN_V, D), lambda j: (0, 0)),
            pl.BlockSpec((SYN_V, 1), lambda j: (0, 0)),
            pl.BlockSpec((emb_d, BN), lambda j: (0, j)),
        ],
        out_specs=[
            pl.BlockSpec((BN, n), lambda j: (j, 0)),
            pl.BlockSpec((n, D), lambda j: (0, 0)),
            pl.BlockSpec((SYN_V, n), lambda j: (0, 0)),
        ],
        out_shape=[
            jax.ShapeDtypeStruct((V, n), jnp.float32),
            jax.ShapeDtypeStruct((n, D), jnp.float32),
            jax.ShapeDtypeStruct((SYN_V, n), jnp.float32),
        ],
        scratch_shapes=[pltpu.VMEM((n, emb_d), jnp.float32)],
        compiler_params=pltpu.CompilerParams(
            vmem_limit_bytes=100 * 1024 * 1024,
            fuse_transposed_lhs_in_matmul=True,
        ),
    )


def kernel(ids, syn_map, table1, table2, W_emb_out, b_emb_out,
           W_sum_out, b_sum_out, W_rev, padding):
    n = ids.shape[0]
    V, D = table1.shape
    SYN_V = W_sum_out.shape[1]
    ACD = padding.shape[1]

    ids = ids.astype(jnp.int32)
    syn_flat = syn_map.astype(jnp.int32).T.reshape(-1)

    sc = _make_sc_kernel(n, V, D, SYN_V)
    g1, g2, sis = sc(ids, syn_flat, table1, table2)

    tc = _make_tc_kernel(n, V, D, ACD, SYN_V, 4096)
    embt, sie, esst = tc(
        g1, g2, W_emb_out, b_emb_out.reshape(1, D),
        W_sum_out.T, b_sum_out.reshape(SYN_V, 1), W_rev.T,
    )
    return embt.T, sie, sis, esst.T


# speedup vs baseline: 3.8682x; 1.0380x over previous
import functools

import jax
import jax.numpy as jnp
from jax import lax
from jax.experimental import pallas as pl
from jax.experimental.pallas import tpu as pltpu
from jax.experimental.pallas import tpu_sc as plsc

_NC = 2
_NS = 16
_NW = _NC * _NS


def _make_sc_gather_kernel(n, V, D):
    rpw = n // _NW
    mesh = plsc.VectorSubcoreMesh(core_axis_name="c", subcore_axis_name="s")

    @functools.partial(
        pl.kernel,
        mesh=mesh,
        out_type=[
            jax.ShapeDtypeStruct((n, D), jnp.float32),
            jax.ShapeDtypeStruct((n, D), jnp.float32),
        ],
        scratch_types=[
            pltpu.VMEM((rpw,), jnp.int32),
            pltpu.VMEM((rpw, D), jnp.float32),
            pltpu.VMEM((rpw, D), jnp.float32),
            pltpu.SemaphoreType.DMA,
            pltpu.SemaphoreType.DMA,
        ],
        compiler_params=pltpu.CompilerParams(needs_layout_passes=False),
    )
    def gather_kernel(ids_hbm, t1_hbm, t2_hbm, g1_hbm, g2_hbm,
                      idx_v, g1_v, g2_v, sem1, sem2):
        wid = lax.axis_index("c") * _NS + lax.axis_index("s")
        base = wid * rpw
        pltpu.sync_copy(ids_hbm.at[pl.ds(base, rpw)], idx_v)
        cp1 = pltpu.async_copy(t1_hbm.at[idx_v], g1_v, sem1)
        cp2 = pltpu.async_copy(t2_hbm.at[idx_v], g2_v, sem2)
        cp1.wait()
        cp2.wait()
        pltpu.sync_copy(g1_v, g1_hbm.at[pl.ds(base, rpw)])
        pltpu.sync_copy(g2_v, g2_hbm.at[pl.ds(base, rpw)])

    return gather_kernel


def _make_sc_scatter_kernel(n, V, SYN_V):
    nw = n // 128
    rpw = 128
    active = nw // _NC
    mesh = plsc.VectorSubcoreMesh(core_axis_name="c", subcore_axis_name="s")

    @functools.partial(
        pl.kernel,
        mesh=mesh,
        out_type=jax.ShapeDtypeStruct((SYN_V, n), jnp.float32),
        scratch_types=[
            pltpu.VMEM((rpw,), jnp.int32),
            pltpu.VMEM((rpw,), jnp.int32),
            pltpu.VMEM((rpw,), jnp.int32),
            pltpu.VMEM((rpw,), jnp.int32),
            pltpu.VMEM((SYN_V, rpw), jnp.float32),
            pltpu.SemaphoreType.DMA,
        ],
        compiler_params=pltpu.CompilerParams(needs_layout_passes=False),
    )
    def scatter_kernel(ids_hbm, syn_flat_hbm, sist_hbm,
                       idx_v, idxo_v, syn0_v, syn1_v, sis_v, sem):
        cid = lax.axis_index("c")
        sid = lax.axis_index("s")

        @pl.when(sid < active)
        def _():
            base = (cid * active + sid) * rpw
            lane = lax.broadcasted_iota(jnp.int32, (16,), 0)
            pltpu.sync_copy(ids_hbm.at[pl.ds(base, rpw)], idx_v)
            cp0 = pltpu.async_copy(syn_flat_hbm.at[idx_v], syn0_v, sem)
            for c in range(rpw // 16):
                idxo_v[pl.ds(c * 16, 16)] = idx_v[pl.ds(c * 16, 16)] + V
            cp1 = pltpu.async_copy(syn_flat_hbm.at[idxo_v], syn1_v, sem)

            zeros16 = jnp.zeros((16,), jnp.float32)

            def zero_body(r, carry):
                for u in range(rpw // 16):
                    sis_v[r, pl.ds(u * 16, 16)] = zeros16
                return carry

            lax.fori_loop(0, SYN_V, zero_body, 0)

            cp0.wait()
            cp1.wait()
            ones16 = jnp.ones((16,), jnp.float32)
            for u in range(rpw // 16):
                cols = lane + 16 * u
                plsc.addupdate_scatter(
                    sis_v, [syn0_v[pl.ds(u * 16, 16)], cols], ones16)
                plsc.addupdate_scatter(
                    sis_v, [syn1_v[pl.ds(u * 16, 16)], cols], ones16)

            pltpu.sync_copy(sis_v, sist_hbm.at[:, pl.ds(base, rpw)])

    return scatter_kernel


def _tc_body(nblk, g1_ref, g2_ref, wemb_ref, bemb_ref, wsumt_ref, bsumt_ref,
             wrevt_ref, embt_ref, sie_ref, esst_ref, x_ref):
    @pl.when(pl.program_id(0) == 0)
    def _():
        g2 = g2_ref[...]
        sie = jax.nn.sigmoid(g2 - 4.0) - jax.nn.sigmoid(-g2 - 4.0)
        sie_ref[...] = sie
        x = (
            g1_ref[...]
            + jnp.dot(sie, wemb_ref[...], preferred_element_type=jnp.float32)
            + bemb_ref[...]
        )
        pad = jnp.full((x.shape[0], 64), 0.1, dtype=jnp.float32)
        x_ref[...] = jnp.concatenate([x, pad], axis=1)

    embt_ref[...] = lax.dot_general(
        wrevt_ref[...], x_ref[...],
        (((0,), (1,)), ((), ())),
        preferred_element_type=jnp.float32,
    )

    @pl.when(pl.program_id(0) == nblk - 1)
    def _():
        esst_ref[...] = (
            lax.dot_general(
                wsumt_ref[...], sie_ref[...],
                (((1,), (1,)), ((), ())),
                preferred_element_type=jnp.float32,
            )
            + bsumt_ref[...]
        )


def _make_tc_kernel(n, V, D, ACD, SYN_V, BN):
    nblk = pl.cdiv(V, BN)
    emb_d = D + ACD
    return pl.pallas_call(
        functools.partial(_tc_body, nblk),
        grid=(nblk,),
        in_specs=[
            pl.BlockSpec((n, D), lambda j: (0, 0)),
            pl.BlockSpec((n, D), lambda j: (0, 0)),
            pl.BlockSpec((D, D), lambda j: (0, 0)),
            pl.BlockSpec((1, D), lambda j: (0, 0)),
            pl.BlockSpec((SYN_V, D), lambda j: (0, 0)),
            pl.BlockSpec((SYN_V, 1), lambda j: (0, 0)),
            pl.BlockSpec((emb_d, BN), lambda j: (0, j)),
        ],
        out_specs=[
            pl.BlockSpec((BN, n), lambda j: (j, 0)),
            pl.BlockSpec((n, D), lambda j: (0, 0)),
            pl.BlockSpec((SYN_V, n), lambda j: (0, 0)),
        ],
        out_shape=[
            jax.ShapeDtypeStruct((V, n), jnp.float32),
            jax.ShapeDtypeStruct((n, D), jnp.float32),
            jax.ShapeDtypeStruct((SYN_V, n), jnp.float32),
        ],
        scratch_shapes=[pltpu.VMEM((n, emb_d), jnp.float32)],
        compiler_params=pltpu.CompilerParams(
            vmem_limit_bytes=100 * 1024 * 1024,
            fuse_transposed_lhs_in_matmul=True,
        ),
    )


def kernel(ids, syn_map, table1, table2, W_emb_out, b_emb_out,
           W_sum_out, b_sum_out, W_rev, padding):
    n = ids.shape[0]
    V, D = table1.shape
    SYN_V = W_sum_out.shape[1]
    ACD = padding.shape[1]

    ids = ids.astype(jnp.int32)
    syn_flat = syn_map.astype(jnp.int32).T.reshape(-1)

    g1, g2 = _make_sc_gather_kernel(n, V, D)(ids, table1, table2)
    sis_t = _make_sc_scatter_kernel(n, V, SYN_V)(ids, syn_flat)

    tc = _make_tc_kernel(n, V, D, ACD, SYN_V, 4096)
    embt, sie, esst = tc(
        g1, g2, W_emb_out, b_emb_out.reshape(1, D),
        W_sum_out.T, b_sum_out.reshape(SYN_V, 1), W_rev.T,
    )
    return embt.T, sie, sis_t.T, esst.T
